# baseline (speedup 1.0000x reference)
import functools

import jax
import jax.numpy as jnp
from jax import lax
from jax.experimental import pallas as pl
from jax.experimental.pallas import tpu as pltpu
from jax.experimental.pallas import tpu_sc as plsc

_NC = 2
_NS = 16
_NW = _NC * _NS
_LANES = 16


def kernel(inputs, embed_weight):
    B, T, D = inputs.shape
    n = T * D
    TPW = T // _NW
    CH = 32
    NCH = TPW // CH
    chunk_elems = CH * D

    x_flat = inputs.reshape(B * T * D)
    w_flat = embed_weight.reshape(n)

    mesh = plsc.VectorSubcoreMesh(core_axis_name="c", subcore_axis_name="s")

    @functools.partial(
        pl.kernel,
        mesh=mesh,
        out_type=jax.ShapeDtypeStruct((B * T * D,), jnp.float32),
        scratch_types=[
            pltpu.VMEM((chunk_elems,), jnp.float32),
            pltpu.VMEM((chunk_elems,), jnp.float32),
        ],
    )
    def sc_add(x_hbm, w_hbm, o_hbm, wbuf, xbuf):
        wid = lax.axis_index("s") * _NC + lax.axis_index("c")
        t0 = wid * TPW

        def add_w_into_x(i, _):
            wv = wbuf[pl.ds(i * _LANES, _LANES)]
            xv = xbuf[pl.ds(i * _LANES, _LANES)]
            xbuf[pl.ds(i * _LANES, _LANES)] = xv + wv
            return _

        for c in range(NCH):
            woff = (t0 + c * CH) * D
            pltpu.sync_copy(w_hbm.at[pl.ds(woff, chunk_elems)], wbuf)
            for b in range(B):
                xoff = (b * T + t0 + c * CH) * D
                pltpu.sync_copy(x_hbm.at[pl.ds(xoff, chunk_elems)], xbuf)
                lax.fori_loop(0, chunk_elems // _LANES, add_w_into_x, 0,
                              unroll=8)
                pltpu.sync_copy(xbuf, o_hbm.at[pl.ds(xoff, chunk_elems)])

    out = sc_add(x_flat, w_flat)
    return out.reshape(B, T, D)


# speedup vs baseline: 1.8985x; 1.8985x over previous
import functools

import jax
import jax.numpy as jnp
from jax import lax
from jax.experimental import pallas as pl
from jax.experimental.pallas import tpu as pltpu
from jax.experimental.pallas import tpu_sc as plsc

_NC = 2
_NS = 16
_NW = _NC * _NS
_LANES = 16
_NBUF = 4


def kernel(inputs, embed_weight):
    B, T, D = inputs.shape
    TPW = T // _NW
    CH = 16
    NCH = TPW // CH
    NJ = NCH * B
    jobs = [(c, b) for c in range(NCH) for b in range(B)]

    x2 = inputs.reshape(B * T, D)

    mesh = plsc.VectorSubcoreMesh(core_axis_name="c", subcore_axis_name="s")

    @functools.partial(
        pl.kernel,
        mesh=mesh,
        out_type=jax.ShapeDtypeStruct((B * T, D), jnp.float32),
        scratch_types=(
            [pltpu.VMEM((CH, D), jnp.float32) for _ in range(2)]
            + [pltpu.VMEM((CH, D), jnp.float32) for _ in range(_NBUF)]
            + [pltpu.SemaphoreType.DMA for _ in range(2 + 2 * _NBUF)]
        ),
    )
    def sc_add(x_hbm, w_hbm, o_hbm, wb0, wb1, xb0, xb1, xb2, xb3,
               ws0, ws1, is0, is1, is2, is3, os0, os1, os2, os3):
        wbufs = (wb0, wb1)
        wsems = (ws0, ws1)
        xbufs = (xb0, xb1, xb2, xb3)
        isems = (is0, is1, is2, is3)
        osems = (os0, os1, os2, os3)
        wid = lax.axis_index("s") * _NC + lax.axis_index("c")
        t0 = wid * TPW

        def xrow(j):
            c, b = jobs[j]
            return b * T + t0 + c * CH

        def start_in(j):
            return pltpu.async_copy(
                x_hbm.at[pl.ds(xrow(j), CH)], xbufs[j % _NBUF],
                isems[j % _NBUF])

        def start_w(c):
            return pltpu.async_copy(
                w_hbm.at[pl.ds(t0 + c * CH, CH)], wbufs[c % 2], wsems[c % 2])

        def add_w_into_x(xbuf, wbuf):
            def row_body(r, _):
                def grp_body(g, _2):
                    sl = pl.ds(g * _LANES, _LANES)
                    xbuf[r, sl] = xbuf[r, sl] + wbuf[r, sl]
                    return _2
                return lax.fori_loop(0, D // _LANES, grp_body, 0, unroll=16)
            lax.fori_loop(0, CH, row_body, 0)

        wcp = [None] * NCH
        incp = [None] * NJ
        outcp = [None] * NJ
        wcp[0] = start_w(0)
        for j in range(min(_NBUF - 1, NJ)):
            incp[j] = start_in(j)
        for j in range(NJ):
            c, b = jobs[j]
            if b == 0:
                if c + 1 < NCH:
                    wcp[c + 1] = start_w(c + 1)
                wcp[c].wait()
            incp[j].wait()
            add_w_into_x(xbufs[j % _NBUF], wbufs[c % 2])
            outcp[j] = pltpu.async_copy(
                xbufs[j % _NBUF], o_hbm.at[pl.ds(xrow(j), CH)],
                osems[j % _NBUF])
            nxt = j + _NBUF - 1
            if nxt < NJ:
                if nxt >= _NBUF:
                    outcp[nxt - _NBUF].wait()
                incp[nxt] = start_in(nxt)
        for j in range(NJ - _NBUF, NJ):
            outcp[j].wait()

    out2 = sc_add(x2, embed_weight)
    return out2.reshape(B, T, D)


# speedup vs baseline: 7.3146x; 3.8528x over previous
import jax
import jax.numpy as jnp
from jax.experimental import pallas as pl


def _add_body(x_ref, w_ref, o_ref):
    t = pl.program_id(0)
    BT = x_ref.shape[0]
    o_ref[...] = x_ref[...] + w_ref[pl.ds(t * BT, BT), :]


def kernel(inputs, embed_weight):
    B, T, D = inputs.shape
    BT = 512
    x2 = inputs.reshape(B * T, D)
    out2 = pl.pallas_call(
        _add_body,
        grid=(T // BT, B),
        in_specs=[
            pl.BlockSpec((BT, D), lambda t, b: (b * (2048 // BT) + t, 0)),
            pl.BlockSpec((T, D), lambda t, b: (0, 0)),
        ],
        out_specs=pl.BlockSpec((BT, D), lambda t, b: (b * (2048 // BT) + t, 0)),
        out_shape=jax.ShapeDtypeStruct((B * T, D), inputs.dtype),
    )(x2, embed_weight)
    return out2.reshape(B, T, D)


# speedup vs baseline: 7.5820x; 1.0366x over previous
import jax
import jax.numpy as jnp
from jax.experimental import pallas as pl
from jax.experimental.pallas import tpu as pltpu

_NBUF = 3


def kernel(inputs, embed_weight):
    B, T, D = inputs.shape
    CH = 512
    NCH = T // CH
    NJ = B * NCH
    jobs = [(b, c) for b in range(B) for c in range(NCH)]

    x2 = inputs.reshape(B * T, D)

    def body(x_hbm, w_hbm, o_hbm, wvm, xb0, xb1, xb2,
             wsem, isem, osem):
        xbufs = (xb0, xb1, xb2)

        def row(j):
            b, c = jobs[j]
            return b * T + c * CH

        def start_w(c):
            cp = pltpu.make_async_copy(
                w_hbm.at[pl.ds(c * CH, CH)], wvm.at[pl.ds(c * CH, CH)],
                wsem.at[c])
            cp.start()
            return cp

        def start_in(j):
            cp = pltpu.make_async_copy(
                x_hbm.at[pl.ds(row(j), CH)], xbufs[j % _NBUF],
                isem.at[j % _NBUF])
            cp.start()
            return cp

        def start_out(j):
            cp = pltpu.make_async_copy(
                xbufs[j % _NBUF], o_hbm.at[pl.ds(row(j), CH)],
                osem.at[j % _NBUF])
            cp.start()
            return cp

        wcps = [start_w(c) for c in range(NCH)]
        incp = [None] * NJ
        outcp = [None] * NJ
        for j in range(_NBUF - 1):
            incp[j] = start_in(j)
        for j in range(NJ):
            b, c = jobs[j]
            if b == 0:
                wcps[c].wait()
            incp[j].wait()
            xbuf = xbufs[j % _NBUF]
            xbuf[...] = xbuf[...] + wvm[pl.ds(c * CH, CH), :]
            outcp[j] = start_out(j)
            nxt = j + _NBUF - 1
            if nxt < NJ:
                if nxt >= _NBUF:
                    outcp[nxt - _NBUF].wait()
                incp[nxt] = start_in(nxt)
        for j in range(NJ - _NBUF, NJ):
            outcp[j].wait()

    out2 = pl.pallas_call(
        body,
        grid=(1,),
        in_specs=[
            pl.BlockSpec(memory_space=pl.ANY),
            pl.BlockSpec(memory_space=pl.ANY),
        ],
        out_specs=pl.BlockSpec(memory_space=pl.ANY),
        out_shape=jax.ShapeDtypeStruct((B * T, D), inputs.dtype),
        scratch_shapes=[
            pltpu.VMEM((T, D), jnp.float32),
            pltpu.VMEM((CH, D), jnp.float32),
            pltpu.VMEM((CH, D), jnp.float32),
            pltpu.VMEM((CH, D), jnp.float32),
            pltpu.SemaphoreType.DMA((NCH,)),
            pltpu.SemaphoreType.DMA((_NBUF,)),
            pltpu.SemaphoreType.DMA((_NBUF,)),
        ],
    )(x2, embed_weight)
    return out2.reshape(B, T, D)


# speedup vs baseline: 8.2024x; 1.0818x over previous
import jax
import jax.numpy as jnp
from jax.experimental import pallas as pl
from jax.experimental.pallas import tpu as pltpu

_NBUF = 3


def kernel(inputs, embed_weight):
    B, T, D = inputs.shape
    CH = 1024
    NCH = T // CH
    NJ = B * NCH
    jobs = [(b, c) for b in range(B) for c in range(NCH)]

    x2 = inputs.reshape(B * T, D)

    def body(x_hbm, w_hbm, o_hbm, wvm, xb0, xb1, xb2,
             wsem, isem, osem):
        xbufs = (xb0, xb1, xb2)

        def row(j):
            b, c = jobs[j]
            return b * T + c * CH

        def start_w(c):
            cp = pltpu.make_async_copy(
                w_hbm.at[pl.ds(c * CH, CH)], wvm.at[pl.ds(c * CH, CH)],
                wsem.at[c])
            cp.start()
            return cp

        def start_in(j):
            cp = pltpu.make_async_copy(
                x_hbm.at[pl.ds(row(j), CH)], xbufs[j % _NBUF],
                isem.at[j % _NBUF])
            cp.start()
            return cp

        def start_out(j):
            cp = pltpu.make_async_copy(
                xbufs[j % _NBUF], o_hbm.at[pl.ds(row(j), CH)],
                osem.at[j % _NBUF])
            cp.start()
            return cp

        wcps = [start_w(c) for c in range(NCH)]
        incp = [None] * NJ
        outcp = [None] * NJ
        for j in range(_NBUF - 1):
            incp[j] = start_in(j)
        for j in range(NJ):
            b, c = jobs[j]
            if b == 0:
                wcps[c].wait()
            incp[j].wait()
            xbuf = xbufs[j % _NBUF]
            xbuf[...] = xbuf[...] + wvm[pl.ds(c * CH, CH), :]
            outcp[j] = start_out(j)
            nxt = j + _NBUF - 1
            if nxt < NJ:
                if nxt >= _NBUF:
                    outcp[nxt - _NBUF].wait()
                incp[nxt] = start_in(nxt)
        for j in range(NJ - _NBUF, NJ):
            outcp[j].wait()

    out2 = pl.pallas_call(
        body,
        grid=(1,),
        in_specs=[
            pl.BlockSpec(memory_space=pl.ANY),
            pl.BlockSpec(memory_space=pl.ANY),
        ],
        out_specs=pl.BlockSpec(memory_space=pl.ANY),
        out_shape=jax.ShapeDtypeStruct((B * T, D), inputs.dtype),
        scratch_shapes=[
            pltpu.VMEM((T, D), jnp.float32),
            pltpu.VMEM((CH, D), jnp.float32),
            pltpu.VMEM((CH, D), jnp.float32),
            pltpu.VMEM((CH, D), jnp.float32),
            pltpu.SemaphoreType.DMA((NCH,)),
            pltpu.SemaphoreType.DMA((_NBUF,)),
            pltpu.SemaphoreType.DMA((_NBUF,)),
        ],
    )(x2, embed_weight)
    return out2.reshape(B, T, D)


# speedup vs baseline: 8.4860x; 1.0346x over previous
import jax
import jax.numpy as jnp
from jax.experimental import pallas as pl
from jax.experimental.pallas import tpu as pltpu

_NBUF = 3


def kernel(inputs, embed_weight):
    B, T, D = inputs.shape
    CH = T
    NJ = B
    x2 = inputs.reshape(B * T, D)

    def body(x_hbm, w_hbm, o_hbm, wvm, xb0, xb1, xb2, wsem, isem, osem):
        xbufs = (xb0, xb1, xb2)

        def start_in(j):
            cp = pltpu.make_async_copy(
                x_hbm.at[pl.ds(j * CH, CH)], xbufs[j % _NBUF],
                isem.at[j % _NBUF])
            cp.start()
            return cp

        def start_out(j):
            cp = pltpu.make_async_copy(
                xbufs[j % _NBUF], o_hbm.at[pl.ds(j * CH, CH)],
                osem.at[j % _NBUF])
            cp.start()
            return cp

        wcp = pltpu.make_async_copy(w_hbm, wvm, wsem)
        wcp.start()
        incp = [None] * NJ
        outcp = [None] * NJ
        for j in range(_NBUF - 1):
            incp[j] = start_in(j)
        wcp.wait()
        for j in range(NJ):
            incp[j].wait()
            xbuf = xbufs[j % _NBUF]
            xbuf[...] = xbuf[...] + wvm[...]
            outcp[j] = start_out(j)
            nxt = j + _NBUF - 1
            if nxt < NJ:
                if nxt >= _NBUF:
                    outcp[nxt - _NBUF].wait()
                incp[nxt] = start_in(nxt)
        for j in range(max(0, NJ - _NBUF), NJ):
            outcp[j].wait()

    out2 = pl.pallas_call(
        body,
        grid=(1,),
        in_specs=[
            pl.BlockSpec(memory_space=pl.ANY),
            pl.BlockSpec(memory_space=pl.ANY),
        ],
        out_specs=pl.BlockSpec(memory_space=pl.ANY),
        out_shape=jax.ShapeDtypeStruct((B * T, D), inputs.dtype),
        scratch_shapes=[
            pltpu.VMEM((T, D), jnp.float32),
            pltpu.VMEM((CH, D), jnp.float32),
            pltpu.VMEM((CH, D), jnp.float32),
            pltpu.VMEM((CH, D), jnp.float32),
            pltpu.SemaphoreType.DMA,
            pltpu.SemaphoreType.DMA((_NBUF,)),
            pltpu.SemaphoreType.DMA((_NBUF,)),
        ],
    )(x2, embed_weight)
    return out2.reshape(B, T, D)


# speedup vs baseline: 8.5839x; 1.0115x over previous
import jax
import jax.numpy as jnp
from jax.experimental import pallas as pl
from jax.experimental.pallas import tpu as pltpu

_NBUF = 3


def kernel(inputs, embed_weight):
    B, T, D = inputs.shape
    H = T // 2
    x2 = inputs.reshape(B * T, D)
    jobs = [(0, H, 0), (H, H, H)]
    for b in range(1, B - 1):
        jobs.append((b * T, T, 0))
    jobs += [((B - 1) * T, H, 0), ((B - 1) * T + H, H, H)]
    NJ = len(jobs)

    def body(x_hbm, w_hbm, o_hbm, wvm, xb0, xb1, xb2, wsem, isem, osem):
        xbufs = (xb0, xb1, xb2)

        def start_in(j):
            r0, n, _ = jobs[j]
            cp = pltpu.make_async_copy(
                x_hbm.at[pl.ds(r0, n)], xbufs[j % _NBUF].at[pl.ds(0, n)],
                isem.at[j % _NBUF])
            cp.start()
            return cp

        def start_out(j):
            r0, n, _ = jobs[j]
            cp = pltpu.make_async_copy(
                xbufs[j % _NBUF].at[pl.ds(0, n)], o_hbm.at[pl.ds(r0, n)],
                osem.at[j % _NBUF])
            cp.start()
            return cp

        wcps = []
        for h in range(2):
            cp = pltpu.make_async_copy(
                w_hbm.at[pl.ds(h * H, H)], wvm.at[pl.ds(h * H, H)],
                wsem.at[h])
            cp.start()
            wcps.append(cp)
        incp = [None] * NJ
        outcp = [None] * NJ
        for j in range(_NBUF - 1):
            incp[j] = start_in(j)
        for j in range(NJ):
            r0, n, woff = jobs[j]
            if j < 2:
                wcps[j].wait()
            incp[j].wait()
            xbuf = xbufs[j % _NBUF]
            xbuf[pl.ds(0, n)] = xbuf[pl.ds(0, n)] + wvm[pl.ds(woff, n)]
            outcp[j] = start_out(j)
            nxt = j + _NBUF - 1
            if nxt < NJ:
                if nxt >= _NBUF:
                    outcp[nxt - _NBUF].wait()
                incp[nxt] = start_in(nxt)
        for j in range(max(0, NJ - _NBUF), NJ):
            outcp[j].wait()

    out2 = pl.pallas_call(
        body,
        grid=(1,),
        in_specs=[
            pl.BlockSpec(memory_space=pl.ANY),
            pl.BlockSpec(memory_space=pl.ANY),
        ],
        out_specs=pl.BlockSpec(memory_space=pl.ANY),
        out_shape=jax.ShapeDtypeStruct((B * T, D), inputs.dtype),
        scratch_shapes=[
            pltpu.VMEM((T, D), jnp.float32),
            pltpu.VMEM((T, D), jnp.float32),
            pltpu.VMEM((T, D), jnp.float32),
            pltpu.VMEM((T, D), jnp.float32),
            pltpu.SemaphoreType.DMA((2,)),
            pltpu.SemaphoreType.DMA((_NBUF,)),
            pltpu.SemaphoreType.DMA((_NBUF,)),
        ],
    )(x2, embed_weight)
    return out2.reshape(B, T, D)
